# baseline (speedup 1.0000x reference)
import functools
import struct

import jax
import jax.numpy as jnp
from jax import lax
from jax.experimental import pallas as pl
from jax.experimental.pallas import tpu as pltpu
from jax.experimental.pallas import tpu_sc as plsc

_GAMMA = 0.999
_P = 0.9995

_L = 16
_NC = 2
_NS = 16
_NW = _NC * _NS
_NBINS = 1 << 16
_HALF = 1 << 15
_CROWS = 8
_CROWS2 = 4
_NRING = 4
_UNROLL = 2

_GBITS = struct.unpack("<i", struct.pack("<f", _GAMMA))[0]
_KEY_G = (_GBITS | 0x80000000) & 0xFFFFFFFF
_HI_G = _KEY_G >> 16
_LO_G = _KEY_G & 0xFFFF


def _zero_hist(hist):
    zeros = jnp.zeros((_L,), jnp.int32)

    @plsc.parallel_loop(0, _NBINS // _L, 1, unroll=8)
    def _(i):
        hist[pl.ds(i * _L, _L)] = zeros


def _hi_body(rows, cols):
    rows_w = rows // _NW
    nchunk = rows_w // _CROWS
    nouter = nchunk // 2
    vec_per_row = cols // _L

    def process(buf, hist):
        ones = jnp.ones((_L,), jnp.int32)
        gam = jnp.float32(_GAMMA)

        def vec(i):
            for r in range(_CROWS):
                v = buf[r, pl.ds(i * _L, _L)]
                m = v < gam
                t = lax.bitcast_convert_type(v, jnp.int32)
                hi = lax.shift_right_logical(t, 16)
                plsc.addupdate_scatter(hist, [hi], ones, mask=m)

        plsc.parallel_loop(0, vec_per_row, 1, unroll=_UNROLL)(vec)

    def body(x_hbm, out_hbm, buf0, buf1, hist, sem0, sem1):
        wid = lax.axis_index("s") * _NC + lax.axis_index("c")
        base = wid * rows_w
        _zero_hist(hist)

        pltpu.async_copy(x_hbm.at[pl.ds(base, _CROWS)], buf0, sem0)

        def outer(h, c):
            s1 = pl.multiple_of(base + (2 * h + 1) * _CROWS, _CROWS)
            pltpu.async_copy(x_hbm.at[pl.ds(s1, _CROWS)], buf1, sem1)
            pltpu.make_async_copy(x_hbm.at[pl.ds(0, _CROWS)], buf0, sem0).wait()
            process(buf0, hist)

            @pl.when(h < nouter - 1)
            def _():
                s0 = pl.multiple_of(base + (2 * h + 2) * _CROWS, _CROWS)
                pltpu.async_copy(x_hbm.at[pl.ds(s0, _CROWS)], buf0, sem0)

            pltpu.make_async_copy(x_hbm.at[pl.ds(0, _CROWS)], buf1, sem1).wait()
            process(buf1, hist)
            return c

        lax.fori_loop(0, nouter, outer, 0)
        pltpu.sync_copy(hist, out_hbm.at[wid])

    return body


def _lo_body(rows, cols):
    rows_w = rows // _NW
    nchunk = rows_w // _CROWS2
    nouter = nchunk // _NRING
    vec_per_row = cols // _L

    def process(buf, hist, cv, rv):
        ones = jnp.ones((_L,), jnp.int32)
        gam = jnp.float32(_GAMMA)

        def vec(i):
            for r in range(_CROWS2):
                v = buf[r, pl.ds(i * _L, _L)]
                tu = lax.bitcast_convert_type(v, jnp.uint32)
                u = tu - cv
                m = u < rv
                ui = lax.bitcast_convert_type(u, jnp.int32)
                plsc.addupdate_scatter(hist, [ui], ones, mask=m)
                buf[r, pl.ds(i * _L, _L)] = jnp.where(v < gam, v, gam)

        plsc.parallel_loop(0, vec_per_row, 1, unroll=_UNROLL)(vec)

    def body(x_hbm, filt_hbm, hist_hbm, out_hbm, buf0, buf1, buf2, buf3,
             filtv, hist, g0, g1, g2, g3, s0, s1, s2, s3):
        wid = lax.axis_index("s") * _NC + lax.axis_index("c")
        base = wid * rows_w
        bufs = (buf0, buf1, buf2, buf3)
        gsems = (g0, g1, g2, g3)
        ssems = (s0, s1, s2, s3)
        _zero_hist(hist)
        pltpu.sync_copy(filt_hbm, filtv)
        cv = lax.bitcast_convert_type(filtv[0, :], jnp.uint32)
        rv = lax.bitcast_convert_type(filtv[1, :], jnp.uint32)

        for j in range(_NRING):
            st = pl.multiple_of(base + j * _CROWS2, _CROWS2)
            pltpu.async_copy(x_hbm.at[pl.ds(st, _CROWS2)], bufs[j], gsems[j])

        def outer(h, c):
            for j in range(_NRING):
                pltpu.make_async_copy(
                    x_hbm.at[pl.ds(0, _CROWS2)], bufs[j], gsems[j]
                ).wait()
                process(bufs[j], hist, cv, rv)
                st = pl.multiple_of(
                    base + (h * _NRING + j) * _CROWS2, _CROWS2
                )
                pltpu.async_copy(bufs[j], out_hbm.at[pl.ds(st, _CROWS2)],
                                 ssems[j])

            @pl.when(h < nouter - 1)
            def _():
                for j in range(_NRING):
                    pltpu.make_async_copy(
                        x_hbm.at[pl.ds(0, _CROWS2)], bufs[j], ssems[j]
                    ).wait()
                    st = pl.multiple_of(
                        base + ((h + 1) * _NRING + j) * _CROWS2, _CROWS2
                    )
                    pltpu.async_copy(x_hbm.at[pl.ds(st, _CROWS2)], bufs[j],
                                     gsems[j])

            return c

        lax.fori_loop(0, nouter, outer, 0)
        for j in range(_NRING):
            pltpu.make_async_copy(
                x_hbm.at[pl.ds(0, _CROWS2)], bufs[j], ssems[j]
            ).wait()
        pltpu.sync_copy(hist, hist_hbm.at[wid])

    return body


@functools.cache
def _make_sc_kernels(rows, cols):
    mesh = plsc.VectorSubcoreMesh(core_axis_name="c", subcore_axis_name="s")
    hist_type = jax.ShapeDtypeStruct((_NW, _NBINS), jnp.int32)
    params = pltpu.CompilerParams(needs_layout_passes=False)
    hist_hi = pl.kernel(
        _hi_body(rows, cols),
        out_type=hist_type,
        mesh=mesh,
        compiler_params=params,
        scratch_types=[
            pltpu.VMEM((_CROWS, cols), jnp.float32),
            pltpu.VMEM((_CROWS, cols), jnp.float32),
            pltpu.VMEM((_NBINS,), jnp.int32),
            pltpu.SemaphoreType.DMA,
            pltpu.SemaphoreType.DMA,
        ],
    )
    hist_lo = pl.kernel(
        _lo_body(rows, cols),
        out_type=(hist_type, jax.ShapeDtypeStruct((rows, cols), jnp.float32)),
        mesh=mesh,
        compiler_params=params,
        scratch_types=[
            pltpu.VMEM((_CROWS2, cols), jnp.float32),
            pltpu.VMEM((_CROWS2, cols), jnp.float32),
            pltpu.VMEM((_CROWS2, cols), jnp.float32),
            pltpu.VMEM((_CROWS2, cols), jnp.float32),
            pltpu.VMEM((2, _L), jnp.int32),
            pltpu.VMEM((_NBINS,), jnp.int32),
            pltpu.SemaphoreType.DMA,
            pltpu.SemaphoreType.DMA,
            pltpu.SemaphoreType.DMA,
            pltpu.SemaphoreType.DMA,
            pltpu.SemaphoreType.DMA,
            pltpu.SemaphoreType.DMA,
            pltpu.SemaphoreType.DMA,
            pltpu.SemaphoreType.DMA,
        ],
    )
    return hist_hi, hist_lo


def _reduce_body(h_ref, o_ref):
    o_ref[...] = jnp.sum(h_ref[...], axis=0, keepdims=True)


def _reduce_rows(h_rows):
    nb = _NBINS // 8
    return pl.pallas_call(
        _reduce_body,
        grid=(8,),
        in_specs=[pl.BlockSpec((_NW, nb), lambda j: (0, j))],
        out_specs=pl.BlockSpec((1, nb), lambda j: (0, j)),
        out_shape=jax.ShapeDtypeStruct((1, _NBINS), jnp.int32),
    )(h_rows)


def kernel(x, scale):
    shp = x.shape
    total = 1
    for s in shp:
        total *= s
    cols = shp[-1]
    rows = total // cols
    x2d = x.reshape(rows, cols)

    hist_hi, hist_lo = _make_sc_kernels(rows, cols)
    k_rank = int(total * _P) - 1
    kp1 = jnp.int32(k_rank + 1)

    h_raw = _reduce_rows(hist_hi(x2d)).reshape(_NBINS)
    h1 = jnp.concatenate([h_raw[_HALF:][::-1], h_raw[:_HALF]])
    c1 = jnp.cumsum(h1)
    gamma_total = jnp.int32(total) - c1[_NBINS - 1]
    bins = lax.iota(jnp.int32, _NBINS)
    c1f = c1 + jnp.where(bins >= _HI_G, gamma_total, 0)
    b = jnp.sum((c1f < kp1).astype(jnp.int32))
    h1b = h1[b] + jnp.where(b == _HI_G, gamma_total, 0)
    r = k_rank - (c1f[b] - h1b)

    pos = b >= jnp.int32(_HALF)
    bb = jnp.where(pos, b ^ jnp.int32(_HALF), jnp.int32(0xFFFF) - b)
    cbase = (bb.astype(jnp.uint32) << 16).astype(jnp.int32)
    rng = jnp.where(
        b == _HI_G,
        jnp.int32(_GBITS - (_HI_G ^ 0x8000) * 65536),
        jnp.int32(1 << 16),
    )
    filt = jnp.stack([
        jnp.full((_L,), cbase, jnp.int32),
        jnp.full((_L,), rng, jnp.int32),
    ])

    h2_rows, out2d = hist_lo(x2d, filt)
    out = out2d.reshape(shp)
    h2_raw = h2_rows.sum(axis=0)
    h2 = jnp.where(pos, h2_raw, h2_raw[::-1])
    h2 = h2.at[_LO_G].add(jnp.where(b == _HI_G, gamma_total, 0))
    c2 = jnp.cumsum(h2)
    rp1 = (r + 1).astype(jnp.int32)
    lo = jnp.sum((c2 < rp1).astype(jnp.int32))

    key_u = (b.astype(jnp.uint32) << 16) | lo.astype(jnp.uint32)
    top = jnp.uint32(0x80000000)
    u = jnp.where(key_u >= top, key_u ^ top, ~key_u)
    val = lax.bitcast_convert_type(u, jnp.float32)
    new_scale = jnp.maximum(val, scale)
    return out, new_scale


# speedup vs baseline: 1.0641x; 1.0641x over previous
import functools
import struct

import jax
import jax.numpy as jnp
from jax import lax
from jax.experimental import pallas as pl
from jax.experimental.pallas import tpu as pltpu
from jax.experimental.pallas import tpu_sc as plsc

_GAMMA = 0.999
_P = 0.9995

_L = 16
_NC = 2
_NS = 16
_NW = _NC * _NS
_NBINS = 1 << 16
_HALF = 1 << 15
_CROWS = 8
_UNROLL = 2

_GBITS = struct.unpack("<i", struct.pack("<f", _GAMMA))[0]
_KEY_G = (_GBITS | 0x80000000) & 0xFFFFFFFF
_HI_G = _KEY_G >> 16
_LO_G = _KEY_G & 0xFFFF


def _zero_hist(hist):
    zeros = jnp.zeros((_L,), jnp.int32)

    @plsc.parallel_loop(0, _NBINS // _L, 1, unroll=8)
    def _(i):
        hist[pl.ds(i * _L, _L)] = zeros


def _hist_body(rows, cols, lo_pass):
    rows_w = rows // _NW
    nchunk = rows_w // _CROWS
    nouter = nchunk // 2
    vec_per_row = cols // _L

    def process(buf, hist, params):
        ones = jnp.ones((_L,), jnp.int32)
        gam = jnp.float32(_GAMMA)

        def vec(i):
            for r in range(_CROWS):
                v = buf[r, pl.ds(i * _L, _L)]
                if lo_pass:
                    cv, rv = params
                    tu = lax.bitcast_convert_type(v, jnp.uint32)
                    u = tu - cv
                    m = u < rv
                    ui = lax.bitcast_convert_type(u, jnp.int32)
                    plsc.addupdate_scatter(hist, [ui], ones, mask=m)
                else:
                    m = v < gam
                    t = lax.bitcast_convert_type(v, jnp.int32)
                    hi = lax.shift_right_logical(t, 16)
                    plsc.addupdate_scatter(hist, [hi], ones, mask=m)

        plsc.parallel_loop(0, vec_per_row, 1, unroll=_UNROLL)(vec)

    def body(x_hbm, out_hbm, buf0, buf1, hist, sem0, sem1, params=None):
        wid = lax.axis_index("s") * _NC + lax.axis_index("c")
        base = wid * rows_w
        _zero_hist(hist)

        pltpu.async_copy(x_hbm.at[pl.ds(base, _CROWS)], buf0, sem0)

        def outer(h, c):
            s1 = pl.multiple_of(base + (2 * h + 1) * _CROWS, _CROWS)
            pltpu.async_copy(x_hbm.at[pl.ds(s1, _CROWS)], buf1, sem1)
            pltpu.make_async_copy(x_hbm.at[pl.ds(0, _CROWS)], buf0, sem0).wait()
            process(buf0, hist, params)

            @pl.when(h < nouter - 1)
            def _():
                s0 = pl.multiple_of(base + (2 * h + 2) * _CROWS, _CROWS)
                pltpu.async_copy(x_hbm.at[pl.ds(s0, _CROWS)], buf0, sem0)

            pltpu.make_async_copy(x_hbm.at[pl.ds(0, _CROWS)], buf1, sem1).wait()
            process(buf1, hist, params)
            return c

        lax.fori_loop(0, nouter, outer, 0)
        pltpu.sync_copy(hist, out_hbm.at[wid])

    if lo_pass:
        def body_lo(x_hbm, filt_hbm, out_hbm, buf0, buf1, filtv, hist, sem0,
                    sem1):
            pltpu.sync_copy(filt_hbm, filtv)
            cv = lax.bitcast_convert_type(filtv[0, :], jnp.uint32)
            rv = lax.bitcast_convert_type(filtv[1, :], jnp.uint32)
            body(x_hbm, out_hbm, buf0, buf1, hist, sem0, sem1,
                 params=(cv, rv))

        return body_lo

    def body_hi(x_hbm, out_hbm, buf0, buf1, hist, sem0, sem1):
        body(x_hbm, out_hbm, buf0, buf1, hist, sem0, sem1)

    return body_hi


@functools.cache
def _make_sc_kernels(rows, cols):
    mesh = plsc.VectorSubcoreMesh(core_axis_name="c", subcore_axis_name="s")
    hist_type = jax.ShapeDtypeStruct((_NW, _NBINS), jnp.int32)
    params = pltpu.CompilerParams(needs_layout_passes=False)
    hist_hi = pl.kernel(
        _hist_body(rows, cols, lo_pass=False),
        out_type=hist_type,
        mesh=mesh,
        compiler_params=params,
        scratch_types=[
            pltpu.VMEM((_CROWS, cols), jnp.float32),
            pltpu.VMEM((_CROWS, cols), jnp.float32),
            pltpu.VMEM((_NBINS,), jnp.int32),
            pltpu.SemaphoreType.DMA,
            pltpu.SemaphoreType.DMA,
        ],
    )
    hist_lo = pl.kernel(
        _hist_body(rows, cols, lo_pass=True),
        out_type=hist_type,
        mesh=mesh,
        compiler_params=params,
        scratch_types=[
            pltpu.VMEM((_CROWS, cols), jnp.float32),
            pltpu.VMEM((_CROWS, cols), jnp.float32),
            pltpu.VMEM((2, _L), jnp.int32),
            pltpu.VMEM((_NBINS,), jnp.int32),
            pltpu.SemaphoreType.DMA,
            pltpu.SemaphoreType.DMA,
        ],
    )
    return hist_hi, hist_lo


def _clamp_body(x_ref, o_ref):
    v = x_ref[...]
    o_ref[...] = jnp.where(v < _GAMMA, v, jnp.float32(_GAMMA))


def _clamp_body2(x_ref, prev_ref, o_ref):
    del prev_ref
    v = x_ref[...]
    o_ref[...] = jnp.where(v < _GAMMA, v, jnp.float32(_GAMMA))


def _clamp_top(x2d, row_hi):
    rows, cols = x2d.shape
    br = 512
    return pl.pallas_call(
        _clamp_body,
        grid=(row_hi // br,),
        in_specs=[pl.BlockSpec((br, cols), lambda i: (i, 0))],
        out_specs=pl.BlockSpec((br, cols), lambda i: (i, 0)),
        out_shape=jax.ShapeDtypeStruct((rows, cols), jnp.float32),
    )(x2d)


def _clamp_bottom(x2d, prev, row_lo):
    rows, cols = x2d.shape
    br = 512
    base = row_lo // br
    return pl.pallas_call(
        _clamp_body2,
        grid=((rows - row_lo) // br,),
        in_specs=[
            pl.BlockSpec((br, cols), lambda i: (i + base, 0)),
            pl.BlockSpec(memory_space=pltpu.MemorySpace.HBM),
        ],
        out_specs=pl.BlockSpec((br, cols), lambda i: (i + base, 0)),
        out_shape=jax.ShapeDtypeStruct((rows, cols), jnp.float32),
        input_output_aliases={1: 0},
    )(x2d, prev)


def kernel(x, scale):
    shp = x.shape
    total = 1
    for s in shp:
        total *= s
    cols = shp[-1]
    rows = total // cols
    x2d = x.reshape(rows, cols)

    split = (rows // 2) // 512 * 512
    out_top = _clamp_top(x2d, split)

    hist_hi, hist_lo = _make_sc_kernels(rows, cols)
    k_rank = int(total * _P) - 1
    kp1 = jnp.int32(k_rank + 1)

    h_raw = hist_hi(x2d).sum(axis=0)
    h1 = jnp.concatenate([h_raw[_HALF:][::-1], h_raw[:_HALF]])
    c1 = jnp.cumsum(h1)
    gamma_total = jnp.int32(total) - c1[_NBINS - 1]
    bins = lax.iota(jnp.int32, _NBINS)
    c1f = c1 + jnp.where(bins >= _HI_G, gamma_total, 0)
    b = jnp.sum((c1f < kp1).astype(jnp.int32))
    h1b = h1[b] + jnp.where(b == _HI_G, gamma_total, 0)
    r = k_rank - (c1f[b] - h1b)

    pos = b >= jnp.int32(_HALF)
    bb = jnp.where(pos, b ^ jnp.int32(_HALF), jnp.int32(0xFFFF) - b)
    cbase = (bb.astype(jnp.uint32) << 16).astype(jnp.int32)
    rng = jnp.where(
        b == _HI_G,
        jnp.int32(_GBITS - (_HI_G ^ 0x8000) * 65536),
        jnp.int32(1 << 16),
    )
    filt = jnp.stack([
        jnp.full((_L,), cbase, jnp.int32),
        jnp.full((_L,), rng, jnp.int32),
    ])

    out_dep, filt = lax.optimization_barrier((out_top, filt))
    out = _clamp_bottom(x2d, out_dep, split).reshape(shp)

    h2_raw = hist_lo(x2d, filt).sum(axis=0)
    h2 = jnp.where(pos, h2_raw, h2_raw[::-1])
    h2 = h2.at[_LO_G].add(jnp.where(b == _HI_G, gamma_total, 0))
    c2 = jnp.cumsum(h2)
    rp1 = (r + 1).astype(jnp.int32)
    lo = jnp.sum((c2 < rp1).astype(jnp.int32))

    key_u = (b.astype(jnp.uint32) << 16) | lo.astype(jnp.uint32)
    top = jnp.uint32(0x80000000)
    u = jnp.where(key_u >= top, key_u ^ top, ~key_u)
    val = lax.bitcast_convert_type(u, jnp.float32)
    new_scale = jnp.maximum(val, scale)
    return out, new_scale
